# baseline (speedup 1.0000x reference)
import functools

import jax
import jax.numpy as jnp
from jax import lax
from jax.experimental import pallas as pl
from jax.experimental.pallas import tpu as pltpu
from jax.experimental.pallas import tpu_sc as plsc

_BIG = 1e30
_BIGTH = 1e29
_NEG = -1e9
_K = 16


def _topk_kernel(q_ref, sT_ref, o_ref, *, nsp, r2, sent, rp, bq):
    q0 = q_ref[0, :, 0:1]
    q1 = q_ref[0, :, 1:2]
    q2 = q_ref[0, :, 2:3]
    s0 = sT_ref[0, 0:1, :]
    s1 = sT_ref[0, 1:2, :]
    s2 = sT_ref[0, 2:3, :]
    def bf(x):
        return x.astype(jnp.bfloat16).astype(jnp.float32)
    qs = bf(q0) * bf(s0) + bf(q1) * bf(s1) + bf(q2) * bf(s2)
    qn = q0 * q0 + q1 * q1 + q2 * q2
    sn = s0 * s0 + s1 * s1 + s2 * s2
    d = qn + sn - 2.0 * qs
    d = jnp.where(d <= r2, d, _BIG)
    lane = lax.broadcasted_iota(jnp.int32, (bq, nsp), 1).astype(jnp.float32)
    cols = []
    for i in range(_K):
        m = jnp.min(d, axis=1, keepdims=True)
        eq = d == m
        cand = jnp.where(eq, lane, jnp.float32(nsp))
        amin = jnp.min(cand, axis=1, keepdims=True)
        if i + 1 < _K:
            d = jnp.where(eq, _BIG, d)
        cols.append(jnp.where(m < _BIGTH, amin, jnp.float32(sent)))
    idx = jnp.concatenate(cols, axis=1).astype(jnp.int32)
    o_ref[0] = idx + pl.program_id(0) * rp


def _topk(qp, sTp, nsp, r2, sent, rp, bq=512):
    bs, nqp, _ = qp.shape
    return pl.pallas_call(
        functools.partial(_topk_kernel, nsp=nsp, r2=r2, sent=sent, rp=rp, bq=bq),
        grid=(bs, nqp // bq),
        in_specs=[
            pl.BlockSpec((1, bq, 3), lambda b, i: (b, i, 0)),
            pl.BlockSpec((1, 8, nsp), lambda b, i: (b, 0, 0)),
        ],
        out_specs=pl.BlockSpec((1, bq, _K), lambda b, i: (b, i, 0)),
        out_shape=jax.ShapeDtypeStruct((bs, nqp, _K), jnp.int32),
    )(qp, sTp)


def _make_sc_gathermax(rows_tab, d_tab, d, nq_tot):
    nw = 32
    cq = 8
    qw = nq_tot // nw
    steps = qw // cq
    assert qw % cq == 0 and steps % 2 == 0 and nq_tot % nw == 0
    mesh = plsc.VectorSubcoreMesh(core_axis_name="c", subcore_axis_name="s")

    @functools.partial(
        pl.kernel,
        mesh=mesh,
        out_type=jax.ShapeDtypeStruct((nq_tot, d), jnp.float32),
        scratch_types=[
            pltpu.VMEM((cq * _K,), jnp.int32),
            pltpu.VMEM((cq * _K,), jnp.int32),
            pltpu.VMEM((cq * _K, d_tab), jnp.float32),
            pltpu.VMEM((cq * _K, d_tab), jnp.float32),
            pltpu.VMEM((cq, d), jnp.float32),
            pltpu.SemaphoreType.DMA,
            pltpu.SemaphoreType.DMA,
        ],
    )
    def gather_max(tab_hbm, idx_hbm, out_hbm, idx0, idx1, rows0, rows1,
                   out_v, sem0, sem1):
        wid = lax.axis_index("s") * 2 + lax.axis_index("c")
        q0 = wid * qw
        idx_v = (idx0, idx1)
        rows_v = (rows0, rows1)
        sems = (sem0, sem1)

        def fire(p, g):
            base = (q0 + g * cq) * _K
            pltpu.sync_copy(idx_hbm.at[pl.ds(base, cq * _K)], idx_v[p])
            pltpu.make_async_copy(tab_hbm.at[idx_v[p]], rows_v[p], sems[p]).start()

        fire(0, 0)
        fire(1, 1)

        def body(g2, _):
            for p in range(2):
                g = g2 * 2 + p
                pltpu.make_async_copy(
                    tab_hbm.at[idx_v[p]], rows_v[p], sems[p]).wait()

                def per_q(qi, _):
                    for v in range(d // 16):
                        acc = rows_v[p][qi * _K, pl.ds(v * 16, 16)]
                        for j in range(1, _K):
                            acc = jnp.maximum(
                                acc, rows_v[p][qi * _K + j, pl.ds(v * 16, 16)])
                        out_v[qi, pl.ds(v * 16, 16)] = acc
                    return 0

                lax.fori_loop(0, cq, per_q, 0, unroll=False)
                pltpu.sync_copy(out_v, out_hbm.at[pl.ds(q0 + g * cq, cq)])
                gn = g + 2
                gn = jnp.where(gn < steps, gn, gn - steps)
                fire(p, gn)
            return 0

        lax.fori_loop(0, steps // 2, body, 0, unroll=False)
        pltpu.make_async_copy(tab_hbm.at[idx0], rows0, sem0).wait()
        pltpu.make_async_copy(tab_hbm.at[idx1], rows1, sem1).wait()

    return gather_max


def _prep1_kernel(fT_ref, x2_ref, wf_ref, wx_ref, b_ref, o_ref, *, ns):
    h = jnp.dot(fT_ref[0], wf_ref[...], preferred_element_type=jnp.float32)
    h = h + jnp.dot(x2_ref[0], wx_ref[...], preferred_element_type=jnp.float32)
    h = h + b_ref[...]
    row = lax.broadcasted_iota(jnp.int32, h.shape, 0)
    o_ref[0] = jnp.where(row < ns, h, _NEG)


def _mid_kernel(h1_ref, x1_ref, w1x_ref, w2f_ref, w2x_ref, b2_ref, o_ref, *, ns):
    pq = jnp.dot(x1_ref[0], w1x_ref[...], preferred_element_type=jnp.float32)
    f1 = jnp.maximum(h1_ref[0] - pq, 0.0)
    h2 = jnp.dot(f1, w2f_ref[...], preferred_element_type=jnp.float32)
    h2 = h2 + jnp.dot(x1_ref[0], w2x_ref[...], preferred_element_type=jnp.float32)
    h2 = h2 + b2_ref[...]
    row = lax.broadcasted_iota(jnp.int32, h2.shape, 0)
    o_ref[0, :, 0:64] = jnp.where(row < ns, h2, _NEG)
    o_ref[0, :, 64:128] = jnp.zeros_like(h2)


def _finish_kernel(h2_ref, x0_ref, w2x_ref, wo_ref, bo_ref, o_ref):
    pq = jnp.dot(x0_ref[0], w2x_ref[...], preferred_element_type=jnp.float32)
    f2 = jnp.maximum(h2_ref[0] - pq, 0.0)
    o_ref[0] = jnp.dot(f2, wo_ref[...], preferred_element_type=jnp.float32) + bo_ref[...]


def _dense_call(body, ins, in_blocks, out_block, out_shape):
    bs = out_shape[0]
    return pl.pallas_call(
        body,
        grid=(bs,),
        in_specs=[
            pl.BlockSpec(blk, (lambda b: (b, 0, 0)) if len(blk) == 3 else
                         (lambda b, _l=len(blk): (0,) * _l))
            for blk in in_blocks
        ],
        out_specs=pl.BlockSpec(out_block, lambda b: (b, 0, 0)),
        out_shape=jax.ShapeDtypeStruct(out_shape, jnp.float32),
    )(*ins)


def _pad_rows(x, n, val):
    return jnp.pad(x, ((0, 0), (0, n - x.shape[1]), (0, 0)), constant_values=val)


def _pad_cols(x, n):
    return jnp.pad(x, ((0, 0), (0, 0), (0, n - x.shape[2])))


def kernel(xyz_0, xyz_1, xyz_2, feats, W1, b1, W2, b2, W_out, b_out):
    bs = xyz_0.shape[0]
    n0, n1, n2 = 10240, 2560, 640
    c_in = feats.shape[1]

    x0p = _pad_rows(xyz_0, n0, 1e3)
    x1p = _pad_rows(xyz_1, n1, 1e3)
    x2p = _pad_rows(xyz_2, n2, 1e3)
    x0p8 = _pad_cols(x0p, 8)
    x1p8 = _pad_cols(x1p, 8)
    x2p8 = _pad_cols(x2p, 8)
    x1T = _pad_rows(jnp.transpose(x1p, (0, 2, 1)), 8, 0.0)
    x2T = _pad_rows(jnp.transpose(x2p, (0, 2, 1)), 8, 0.0)
    fT = _pad_rows(jnp.transpose(feats, (0, 2, 1)), n2, 0.0)
    w1fT = W1[:, :c_in].T
    w1xT8 = jnp.pad(W1[:, c_in:].T, ((0, 5), (0, 0)))
    w2fT = W2[:, :128].T
    w2xT8 = jnp.pad(W2[:, 128:].T, ((0, 5), (0, 0)))
    woT = W_out.T
    b1r = b1[None, :]
    b2r = b2[None, :]
    bor = b_out[None, :]

    h1 = _dense_call(
        functools.partial(_prep1_kernel, ns=xyz_2.shape[1]),
        (fT, x2p8, w1fT, w1xT8, b1r),
        [(1, n2, c_in), (1, n2, 8), (c_in, 128), (8, 128), (1, 128)],
        (1, n2, 128), (bs, n2, 128))

    idx1 = _topk(x1p, x2T, n2, 0.5 ** 2, xyz_2.shape[1], n2)
    idx2 = _topk(x0p, x1T, n1, 0.35 ** 2, xyz_1.shape[1], n1)

    g1 = _make_sc_gathermax(bs * n2, 128, 128, bs * n1)(
        h1.reshape(bs * n2, 128), idx1.reshape(-1))
    H1 = g1.reshape(bs, n1, 128)

    h2 = _dense_call(
        functools.partial(_mid_kernel, ns=xyz_1.shape[1]),
        (H1, x1p8, w1xT8, w2fT, w2xT8, b2r),
        [(1, n1, 128), (1, n1, 8), (8, 128), (128, 64), (8, 64), (1, 64)],
        (1, n1, 128), (bs, n1, 128))

    g2 = _make_sc_gathermax(bs * n1, 128, 64, bs * n0)(
        h2.reshape(bs * n1, 128), idx2.reshape(-1))
    H2 = g2.reshape(bs, n0, 64)

    rgb = _dense_call(
        _finish_kernel,
        (H2, x0p8, w2xT8, woT, bor),
        [(1, n0, 64), (1, n0, 8), (8, 64), (64, 3), (1, 3)],
        (1, n0, 3), (bs, n0, 3))

    return jnp.transpose(rgb[:, :xyz_0.shape[1], :], (0, 2, 1))


# speedup vs baseline: 1.1239x; 1.1239x over previous
import functools

import jax
import jax.numpy as jnp
from jax import lax
from jax.experimental import pallas as pl
from jax.experimental.pallas import tpu as pltpu
from jax.experimental.pallas import tpu_sc as plsc

_BIG = 1e30
_BIGTH = 1e29
_NEG = -1e9
_K = 16


def _topk_kernel(q_ref, sT_ref, o_ref, *, nsp, r2, sent, rp, bq):
    q0 = q_ref[0, :, 0:1]
    q1 = q_ref[0, :, 1:2]
    q2 = q_ref[0, :, 2:3]
    s0 = sT_ref[0, 0:1, :]
    s1 = sT_ref[0, 1:2, :]
    s2 = sT_ref[0, 2:3, :]
    def bf(x):
        return x.astype(jnp.bfloat16).astype(jnp.float32)
    qs = bf(q0) * bf(s0) + bf(q1) * bf(s1) + bf(q2) * bf(s2)
    qn = q0 * q0 + q1 * q1 + q2 * q2
    sn = s0 * s0 + s1 * s1 + s2 * s2
    d = qn + sn - 2.0 * qs
    d = jnp.where(d <= r2, d, _BIG)
    lane = lax.broadcasted_iota(jnp.int32, (bq, nsp), 1).astype(jnp.float32)
    cols = []
    for i in range(_K):
        m = jnp.min(d, axis=1, keepdims=True)
        eq = d == m
        cand = jnp.where(eq, lane, jnp.float32(nsp))
        amin = jnp.min(cand, axis=1, keepdims=True)
        if i + 1 < _K:
            d = jnp.where(eq, _BIG, d)
        cols.append(jnp.where(m < _BIGTH, amin, jnp.float32(sent)))
    idx = jnp.concatenate(cols, axis=1).astype(jnp.int32)
    o_ref[0] = idx + pl.program_id(0) * rp


def _topk(qp, sTp, nsp, r2, sent, rp, bq=512):
    bs, nqp, _ = qp.shape
    return pl.pallas_call(
        functools.partial(_topk_kernel, nsp=nsp, r2=r2, sent=sent, rp=rp, bq=bq),
        grid=(bs, nqp // bq),
        in_specs=[
            pl.BlockSpec((1, bq, 3), lambda b, i: (b, i, 0)),
            pl.BlockSpec((1, 8, nsp), lambda b, i: (b, 0, 0)),
        ],
        out_specs=pl.BlockSpec((1, bq, _K), lambda b, i: (b, i, 0)),
        out_shape=jax.ShapeDtypeStruct((bs, nqp, _K), jnp.int32),
    )(qp, sTp)


def _make_sc_gathermax(rows_tab, d_tab, d, nq_tot):
    nw = 32
    cq = 8
    qw = nq_tot // nw
    steps = qw // cq
    assert qw % cq == 0 and steps % 2 == 0 and nq_tot % nw == 0
    mesh = plsc.VectorSubcoreMesh(core_axis_name="c", subcore_axis_name="s")

    @functools.partial(
        pl.kernel,
        mesh=mesh,
        compiler_params=pltpu.CompilerParams(use_tc_tiling_on_sc=False),
        out_type=jax.ShapeDtypeStruct((nq_tot, d), jnp.float32),
        scratch_types=[
            pltpu.VMEM((cq * _K,), jnp.int32),
            pltpu.VMEM((cq * _K,), jnp.int32),
            pltpu.VMEM((cq * _K, d_tab), jnp.float32),
            pltpu.VMEM((cq * _K, d_tab), jnp.float32),
            pltpu.VMEM((cq, d), jnp.float32),
            pltpu.SemaphoreType.DMA,
            pltpu.SemaphoreType.DMA,
        ],
    )
    def gather_max(tab_hbm, idx_hbm, out_hbm, idx0, idx1, rows0, rows1,
                   out_v, sem0, sem1):
        wid = lax.axis_index("s") * 2 + lax.axis_index("c")
        q0 = wid * qw
        idx_v = (idx0, idx1)
        rows_v = (rows0, rows1)
        sems = (sem0, sem1)

        def fire(p, g):
            base = (q0 + g * cq) * _K
            pltpu.sync_copy(idx_hbm.at[pl.ds(base, cq * _K)], idx_v[p])
            pltpu.make_async_copy(tab_hbm.at[idx_v[p]], rows_v[p], sems[p]).start()

        fire(0, 0)
        fire(1, 1)

        def body(g2, _):
            for p in range(2):
                g = g2 * 2 + p
                pltpu.make_async_copy(
                    tab_hbm.at[idx_v[p]], rows_v[p], sems[p]).wait()

                def per_q(qi, _):
                    for v in range(d // 16):
                        acc = rows_v[p][qi * _K, pl.ds(v * 16, 16)]
                        for j in range(1, _K):
                            acc = jnp.maximum(
                                acc, rows_v[p][qi * _K + j, pl.ds(v * 16, 16)])
                        out_v[qi, pl.ds(v * 16, 16)] = acc
                    return 0

                lax.fori_loop(0, cq, per_q, 0, unroll=False)
                pltpu.sync_copy(out_v, out_hbm.at[pl.ds(q0 + g * cq, cq)])
                gn = g + 2
                gn = jnp.where(gn < steps, gn, gn - steps)
                fire(p, gn)
            return 0

        lax.fori_loop(0, steps // 2, body, 0, unroll=False)
        pltpu.make_async_copy(tab_hbm.at[idx0], rows0, sem0).wait()
        pltpu.make_async_copy(tab_hbm.at[idx1], rows1, sem1).wait()

    return gather_max


def _prep1_kernel(fT_ref, x2_ref, wf_ref, wx_ref, b_ref, o_ref, *, ns):
    h = jnp.dot(fT_ref[0], wf_ref[...], preferred_element_type=jnp.float32)
    h = h + jnp.dot(x2_ref[0], wx_ref[...], preferred_element_type=jnp.float32)
    h = h + b_ref[...]
    row = lax.broadcasted_iota(jnp.int32, h.shape, 0)
    o_ref[0] = jnp.where(row < ns, h, _NEG)


def _mid_kernel(h1_ref, x1_ref, w1x_ref, w2f_ref, w2x_ref, b2_ref, o_ref, *, ns, dh):
    pq = jnp.dot(x1_ref[0], w1x_ref[...], preferred_element_type=jnp.float32)
    f1 = jnp.maximum(h1_ref[0] - pq, 0.0)
    h2 = jnp.dot(f1, w2f_ref[...], preferred_element_type=jnp.float32)
    h2 = h2 + jnp.dot(x1_ref[0], w2x_ref[...], preferred_element_type=jnp.float32)
    h2 = h2 + b2_ref[...]
    row = lax.broadcasted_iota(jnp.int32, h2.shape, 0)
    lane = lax.broadcasted_iota(jnp.int32, h2.shape, 1)
    pr = row * 2 + jnp.where(lane >= dh, 1, 0)
    o_ref[0] = jnp.where(pr < ns, h2, _NEG)


def _finish_kernel(h2_ref, x0_ref, w2x_ref, wo_ref, bo_ref, o_ref):
    pq = jnp.dot(x0_ref[0], w2x_ref[...], preferred_element_type=jnp.float32)
    f2 = jnp.maximum(h2_ref[0] - pq, 0.0)
    o_ref[0] = jnp.dot(f2, wo_ref[...], preferred_element_type=jnp.float32) + bo_ref[...]


def _dense_call(body, ins, in_blocks, out_block, out_shape):
    bs = out_shape[0]
    return pl.pallas_call(
        body,
        grid=(bs,),
        in_specs=[
            pl.BlockSpec(blk, (lambda b: (b, 0, 0)) if len(blk) == 3 else
                         (lambda b, _l=len(blk): (0,) * _l))
            for blk in in_blocks
        ],
        out_specs=pl.BlockSpec(out_block, lambda b: (b, 0, 0)),
        out_shape=jax.ShapeDtypeStruct(out_shape, jnp.float32),
    )(*ins)


def _pad_rows(x, n, val):
    return jnp.pad(x, ((0, 0), (0, n - x.shape[1]), (0, 0)), constant_values=val)


def _pad_cols(x, n):
    return jnp.pad(x, ((0, 0), (0, 0), (0, n - x.shape[2])))


def kernel(xyz_0, xyz_1, xyz_2, feats, W1, b1, W2, b2, W_out, b_out):
    bs = xyz_0.shape[0]
    n0, n1, n2 = 10240, 2560, 640
    c_in = feats.shape[1]

    x0p = _pad_rows(xyz_0, n0, 1e3)
    x1p = _pad_rows(xyz_1, n1, 1e3)
    x2p = _pad_rows(xyz_2, n2, 1e3)
    x0p8 = _pad_cols(x0p, 8)
    x1p8 = _pad_cols(x1p, 8)
    x2p8 = _pad_cols(x2p, 8)
    x1T = _pad_rows(jnp.transpose(x1p, (0, 2, 1)), 8, 0.0)
    x2T = _pad_rows(jnp.transpose(x2p, (0, 2, 1)), 8, 0.0)
    fT = _pad_rows(jnp.transpose(feats, (0, 2, 1)), n2, 0.0)
    w1fT = W1[:, :c_in].T
    w1xT8 = jnp.pad(W1[:, c_in:].T, ((0, 5), (0, 0)))
    w2fT = W2[:, :128].T
    w2xT8 = jnp.pad(W2[:, 128:].T, ((0, 5), (0, 0)))
    woT = W_out.T
    b1r = b1[None, :]

    def bd(w):
        z = jnp.zeros_like(w)
        return jnp.concatenate(
            [jnp.concatenate([w, z], 1), jnp.concatenate([z, w], 1)], 0)

    w1x_bd = bd(w1xT8)
    w2f_bd = bd(w2fT)
    w2x_bd = bd(w2xT8)
    wo_bd = bd(woT)
    b2bd = jnp.tile(b2[None, :], (1, 2))
    bobd = jnp.tile(b_out[None, :], (1, 2))
    x1p16 = x1p8.reshape(bs, n1 // 2, 16)
    x0p16 = x0p8.reshape(bs, n0 // 2, 16)

    h1 = _dense_call(
        functools.partial(_prep1_kernel, ns=xyz_2.shape[1]),
        (fT, x2p8, w1fT, w1xT8, b1r),
        [(1, n2, c_in), (1, n2, 8), (c_in, 128), (8, 128), (1, 128)],
        (1, n2, 128), (bs, n2, 128))

    idx1 = _topk(x1p, x2T, n2, 0.5 ** 2, xyz_2.shape[1], n2)
    idx2 = _topk(x0p, x1T, n1, 0.35 ** 2, xyz_1.shape[1], n1)

    g1 = _make_sc_gathermax(bs * n2, 128, 128, bs * n1)(
        h1.reshape(bs * n2, 128), idx1.reshape(-1))
    H1p = g1.reshape(bs, n1 // 2, 256)

    h2 = _dense_call(
        functools.partial(_mid_kernel, ns=xyz_1.shape[1], dh=64),
        (H1p, x1p16, w1x_bd, w2f_bd, w2x_bd, b2bd),
        [(1, n1 // 2, 256), (1, n1 // 2, 16), (16, 256), (256, 128),
         (16, 128), (1, 128)],
        (1, n1 // 2, 128), (bs, n1 // 2, 128))

    g2 = _make_sc_gathermax(bs * n1, 64, 64, bs * n0)(
        h2.reshape(bs * n1, 64), idx2.reshape(-1))
    H2p = g2.reshape(bs, n0 // 2, 128)

    rgb = _dense_call(
        _finish_kernel,
        (H2p, x0p16, w2x_bd, wo_bd, bobd),
        [(1, n0 // 2, 128), (1, n0 // 2, 16), (16, 128), (128, 6), (1, 6)],
        (1, n0 // 2, 6), (bs, n0 // 2, 6))

    rgb = rgb.reshape(bs, n0, 3)
    return jnp.transpose(rgb[:, :xyz_0.shape[1], :], (0, 2, 1))


# speedup vs baseline: 1.2532x; 1.1150x over previous
import functools

import jax
import jax.numpy as jnp
from jax import lax
from jax.experimental import pallas as pl
from jax.experimental.pallas import tpu as pltpu
from jax.experimental.pallas import tpu_sc as plsc

_BIG = 1e30
_BIGTH = 1e29
_NEG = -1e9
_K = 16


def _topk_kernel(q_ref, sT_ref, o_ref, *, nsp, r2, sent, rp, bq):
    q0 = q_ref[0, :, 0:1]
    q1 = q_ref[0, :, 1:2]
    q2 = q_ref[0, :, 2:3]
    s0 = sT_ref[0, 0:1, :]
    s1 = sT_ref[0, 1:2, :]
    s2 = sT_ref[0, 2:3, :]
    qs = jnp.dot(q_ref[0], sT_ref[0, 0:3, :],
                 precision=lax.Precision.DEFAULT,
                 preferred_element_type=jnp.float32)
    qn = q0 * q0 + q1 * q1 + q2 * q2
    sn = s0 * s0 + s1 * s1 + s2 * s2
    d = qn + sn - 2.0 * qs
    d = jnp.where(d <= r2, d, _BIG)
    lane = lax.broadcasted_iota(jnp.int32, (bq, nsp), 1).astype(jnp.float32)
    cols = []
    for i in range(_K):
        m = jnp.min(d, axis=1, keepdims=True)
        cand = jnp.where(d == m, lane, jnp.float32(nsp))
        amin = jnp.min(cand, axis=1, keepdims=True)
        if i + 1 < _K:
            d = jnp.where(lane == amin, _BIG, d)
        cols.append(jnp.where(m < _BIGTH, amin, jnp.float32(sent)))
    idx = jnp.concatenate(cols, axis=1).astype(jnp.int32)
    o_ref[0] = idx + pl.program_id(0) * rp


def _topk(qp, sTp, nsp, r2, sent, rp, bq=512):
    bs, nqp, _ = qp.shape
    return pl.pallas_call(
        functools.partial(_topk_kernel, nsp=nsp, r2=r2, sent=sent, rp=rp, bq=bq),
        grid=(bs, nqp // bq),
        in_specs=[
            pl.BlockSpec((1, bq, 3), lambda b, i: (b, i, 0)),
            pl.BlockSpec((1, 8, nsp), lambda b, i: (b, 0, 0)),
        ],
        out_specs=pl.BlockSpec((1, bq, _K), lambda b, i: (b, i, 0)),
        out_shape=jax.ShapeDtypeStruct((bs, nqp, _K), jnp.int32),
    )(qp, sTp)


def _make_sc_gathermax(rows_tab, d_tab, d, nq_tot):
    nw = 32
    cq = 8
    qw = nq_tot // nw
    steps = qw // cq
    assert qw % cq == 0 and steps % 2 == 0 and nq_tot % nw == 0
    mesh = plsc.VectorSubcoreMesh(core_axis_name="c", subcore_axis_name="s")

    @functools.partial(
        pl.kernel,
        mesh=mesh,
        compiler_params=pltpu.CompilerParams(use_tc_tiling_on_sc=False),
        out_type=jax.ShapeDtypeStruct((nq_tot, d), jnp.float32),
        scratch_types=[
            pltpu.VMEM((cq * _K,), jnp.int32),
            pltpu.VMEM((cq * _K,), jnp.int32),
            pltpu.VMEM((cq * _K, d_tab), jnp.float32),
            pltpu.VMEM((cq * _K, d_tab), jnp.float32),
            pltpu.VMEM((cq, d), jnp.float32),
            pltpu.SemaphoreType.DMA,
            pltpu.SemaphoreType.DMA,
        ],
    )
    def gather_max(tab_hbm, idx_hbm, out_hbm, idx0, idx1, rows0, rows1,
                   out_v, sem0, sem1):
        wid = lax.axis_index("s") * 2 + lax.axis_index("c")
        q0 = wid * qw
        idx_v = (idx0, idx1)
        rows_v = (rows0, rows1)
        sems = (sem0, sem1)

        def fire(p, g):
            base = (q0 + g * cq) * _K
            pltpu.sync_copy(idx_hbm.at[pl.ds(base, cq * _K)], idx_v[p])
            pltpu.make_async_copy(tab_hbm.at[idx_v[p]], rows_v[p], sems[p]).start()

        fire(0, 0)
        fire(1, 1)

        def body(g2, _):
            for p in range(2):
                g = g2 * 2 + p
                pltpu.make_async_copy(
                    tab_hbm.at[idx_v[p]], rows_v[p], sems[p]).wait()

                def per_q(qi, _):
                    for v in range(d // 16):
                        acc = rows_v[p][qi * _K, pl.ds(v * 16, 16)]
                        for j in range(1, _K):
                            acc = jnp.maximum(
                                acc, rows_v[p][qi * _K + j, pl.ds(v * 16, 16)])
                        out_v[qi, pl.ds(v * 16, 16)] = acc
                    return 0

                lax.fori_loop(0, cq, per_q, 0, unroll=False)
                pltpu.sync_copy(out_v, out_hbm.at[pl.ds(q0 + g * cq, cq)])
                gn = g + 2
                gn = jnp.where(gn < steps, gn, gn - steps)
                fire(p, gn)
            return 0

        lax.fori_loop(0, steps // 2, body, 0, unroll=False)
        pltpu.make_async_copy(tab_hbm.at[idx0], rows0, sem0).wait()
        pltpu.make_async_copy(tab_hbm.at[idx1], rows1, sem1).wait()

    return gather_max


def _prep1_kernel(fT_ref, x2_ref, wf_ref, wx_ref, b_ref, o_ref, *, ns):
    h = jnp.dot(fT_ref[0], wf_ref[...], preferred_element_type=jnp.float32)
    h = h + jnp.dot(x2_ref[0], wx_ref[...], preferred_element_type=jnp.float32)
    h = h + b_ref[...]
    row = lax.broadcasted_iota(jnp.int32, h.shape, 0)
    o_ref[0] = jnp.where(row < ns, h, _NEG)


def _mid_kernel(h1_ref, x1_ref, w1x_ref, w2f_ref, w2x_ref, b2_ref, o_ref, *, ns, dh):
    pq = jnp.dot(x1_ref[0], w1x_ref[...], preferred_element_type=jnp.float32)
    f1 = jnp.maximum(h1_ref[0] - pq, 0.0)
    h2 = jnp.dot(f1, w2f_ref[...], preferred_element_type=jnp.float32)
    h2 = h2 + jnp.dot(x1_ref[0], w2x_ref[...], preferred_element_type=jnp.float32)
    h2 = h2 + b2_ref[...]
    row = lax.broadcasted_iota(jnp.int32, h2.shape, 0)
    lane = lax.broadcasted_iota(jnp.int32, h2.shape, 1)
    pr = row * 2 + jnp.where(lane >= dh, 1, 0)
    o_ref[0] = jnp.where(pr < ns, h2, _NEG)


def _finish_kernel(h2_ref, x0_ref, w2x_ref, wo_ref, bo_ref, o_ref):
    pq = jnp.dot(x0_ref[0], w2x_ref[...], preferred_element_type=jnp.float32)
    f2 = jnp.maximum(h2_ref[0] - pq, 0.0)
    o_ref[0] = jnp.dot(f2, wo_ref[...], preferred_element_type=jnp.float32) + bo_ref[...]


def _dense_call(body, ins, in_blocks, out_block, out_shape):
    bs = out_shape[0]
    return pl.pallas_call(
        body,
        grid=(bs,),
        in_specs=[
            pl.BlockSpec(blk, (lambda b: (b, 0, 0)) if len(blk) == 3 else
                         (lambda b, _l=len(blk): (0,) * _l))
            for blk in in_blocks
        ],
        out_specs=pl.BlockSpec(out_block, lambda b: (b, 0, 0)),
        out_shape=jax.ShapeDtypeStruct(out_shape, jnp.float32),
    )(*ins)


def _pad_rows(x, n, val):
    return jnp.pad(x, ((0, 0), (0, n - x.shape[1]), (0, 0)), constant_values=val)


def _pad_cols(x, n):
    return jnp.pad(x, ((0, 0), (0, 0), (0, n - x.shape[2])))


def kernel(xyz_0, xyz_1, xyz_2, feats, W1, b1, W2, b2, W_out, b_out):
    bs = xyz_0.shape[0]
    n0, n1, n2 = 10240, 2560, 640
    c_in = feats.shape[1]

    x0p = _pad_rows(xyz_0, n0, 1e3)
    x1p = _pad_rows(xyz_1, n1, 1e3)
    x2p = _pad_rows(xyz_2, n2, 1e3)
    x0p8 = _pad_cols(x0p, 8)
    x1p8 = _pad_cols(x1p, 8)
    x2p8 = _pad_cols(x2p, 8)
    x1T = _pad_rows(jnp.transpose(x1p, (0, 2, 1)), 8, 0.0)
    x2T = _pad_rows(jnp.transpose(x2p, (0, 2, 1)), 8, 0.0)
    fT = _pad_rows(jnp.transpose(feats, (0, 2, 1)), n2, 0.0)
    w1fT = W1[:, :c_in].T
    w1xT8 = jnp.pad(W1[:, c_in:].T, ((0, 5), (0, 0)))
    w2fT = W2[:, :128].T
    w2xT8 = jnp.pad(W2[:, 128:].T, ((0, 5), (0, 0)))
    woT = W_out.T
    b1r = b1[None, :]

    def bd(w):
        z = jnp.zeros_like(w)
        return jnp.concatenate(
            [jnp.concatenate([w, z], 1), jnp.concatenate([z, w], 1)], 0)

    w1x_bd = bd(w1xT8)
    w2f_bd = bd(w2fT)
    w2x_bd = bd(w2xT8)
    wo_bd = bd(woT)
    b2bd = jnp.tile(b2[None, :], (1, 2))
    bobd = jnp.tile(b_out[None, :], (1, 2))
    x1p16 = x1p8.reshape(bs, n1 // 2, 16)
    x0p16 = x0p8.reshape(bs, n0 // 2, 16)

    h1 = _dense_call(
        functools.partial(_prep1_kernel, ns=xyz_2.shape[1]),
        (fT, x2p8, w1fT, w1xT8, b1r),
        [(1, n2, c_in), (1, n2, 8), (c_in, 128), (8, 128), (1, 128)],
        (1, n2, 128), (bs, n2, 128))

    idx1 = _topk(x1p, x2T, n2, 0.5 ** 2, xyz_2.shape[1], n2)
    idx2 = _topk(x0p, x1T, n1, 0.35 ** 2, xyz_1.shape[1], n1)

    g1 = _make_sc_gathermax(bs * n2, 128, 128, bs * n1)(
        h1.reshape(bs * n2, 128), idx1.reshape(-1))
    H1p = g1.reshape(bs, n1 // 2, 256)

    h2 = _dense_call(
        functools.partial(_mid_kernel, ns=xyz_1.shape[1], dh=64),
        (H1p, x1p16, w1x_bd, w2f_bd, w2x_bd, b2bd),
        [(1, n1 // 2, 256), (1, n1 // 2, 16), (16, 256), (256, 128),
         (16, 128), (1, 128)],
        (1, n1 // 2, 128), (bs, n1 // 2, 128))

    g2 = _make_sc_gathermax(bs * n1, 64, 64, bs * n0)(
        h2.reshape(bs * n1, 64), idx2.reshape(-1))
    H2p = g2.reshape(bs, n0 // 2, 128)

    rgb = _dense_call(
        _finish_kernel,
        (H2p, x0p16, w2x_bd, wo_bd, bobd),
        [(1, n0 // 2, 128), (1, n0 // 2, 16), (16, 128), (128, 6), (1, 6)],
        (1, n0 // 2, 6), (bs, n0 // 2, 6))

    rgb = rgb.reshape(bs, n0, 3)
    return jnp.transpose(rgb[:, :xyz_0.shape[1], :], (0, 2, 1))
